# baseline (speedup 1.0000x reference)
import functools

import jax
import jax.numpy as jnp
from jax import lax
from jax.experimental import pallas as pl
from jax.experimental.pallas import tpu as pltpu
from jax.experimental.pallas import tpu_sc as plsc

BLK_T = 512
BM = 512

_SC_NC = 2
_SC_NS = 16
_SC_NW = _SC_NC * _SC_NS


def _route_body(x_ref, wr_ref, br_ref, tril_ref, dest_ref, be_ref,
                oh_s, counts_s, running_s):
    p = pl.program_id(0)
    i = pl.program_id(1)

    @pl.when(p == 0)
    def _router():
        @pl.when(i == 0)
        def _():
            counts_s[...] = jnp.zeros_like(counts_s)
            running_s[...] = jnp.zeros_like(running_s)

        xb = x_ref[...]
        logits = jnp.dot(xb, wr_ref[...], preferred_element_type=jnp.float32)
        logits = logits + br_ref[...][None, :]
        n, e = logits.shape
        maxv = jnp.max(logits, axis=1, keepdims=True)
        ohm = (logits == maxv).astype(jnp.float32)
        tri8 = (lax.broadcasted_iota(jnp.int32, (e, e), 0)
                < lax.broadcasted_iota(jnp.int32, (e, e), 1)
                ).astype(jnp.float32)
        prior = jnp.dot(ohm, tri8, preferred_element_type=jnp.float32)
        oh = ohm * (prior == 0.0).astype(jnp.float32)
        oh_s[i] = oh
        counts_s[...] += jnp.sum(oh, axis=0, keepdims=True)

    @pl.when(p == 1)
    def _plan():
        counts = counts_s[...]
        e = counts.shape[1]
        pc = jnp.ceil(counts / BM) * BM
        tri = (lax.broadcasted_iota(jnp.int32, (e, e), 0)
               < lax.broadcasted_iota(jnp.int32, (e, e), 1)
               ).astype(jnp.float32)
        offs = jnp.dot(pc, tri, preferred_element_type=jnp.float32)
        ends = offs + pc

        oh = oh_s[i]
        earlier = jnp.dot(tril_ref[...], oh, preferred_element_type=jnp.float32)
        base = offs + running_s[...]
        slot = (base + earlier) * oh
        ones_row = jnp.ones((1, slot.shape[1]), jnp.float32)
        destf = jnp.dot(ones_row, jnp.swapaxes(slot, 0, 1),
                        preferred_element_type=jnp.float32)
        dest_ref[0, :, :] = destf.astype(jnp.int32)
        running_s[...] += jnp.sum(oh, axis=0, keepdims=True)

        @pl.when(i == 0)
        def _():
            jb = lax.broadcasted_iota(jnp.int32, (1, 128), 1)
            jbf = jb.astype(jnp.float32) * BM
            lane8 = lax.broadcasted_iota(jnp.int32, (1, e), 1)
            acc = jnp.zeros((1, 128), jnp.float32)
            for ei in range(e):
                end_e = jnp.sum(jnp.where(lane8 == ei, ends, 0.0))
                acc += (jbf >= end_e).astype(jnp.float32)
            be_ref[...] = acc.astype(jnp.int32)


def _gmm_body(n_experts, be_ref, xs_ref, wfc_ref, bfc_ref, wg_ref, bg_ref,
              wp_ref, bp_ref, out_ref):
    i = pl.program_id(0)

    @pl.when(be_ref[i] < n_experts)
    def _():
        xb = xs_ref[...]
        h = jnp.dot(xb, wfc_ref[0], preferred_element_type=jnp.float32)
        h = h + bfc_ref[0]
        g = jnp.dot(xb, wg_ref[0], preferred_element_type=jnp.float32)
        g = g + bg_ref[0]
        g = g * (1.0 / (1.0 + jnp.exp(-g)))
        o = jnp.dot(h * g, wp_ref[0], preferred_element_type=jnp.float32)
        out_ref[...] = o + bp_ref[0]


def _sc_scatter(xf, dest, npad):
    tok, c = xf.shape
    per_w = tok // _SC_NW
    ch = 64
    n_ch = per_w // ch
    mesh = plsc.VectorSubcoreMesh(core_axis_name="c", subcore_axis_name="s")

    @functools.partial(
        pl.kernel, mesh=mesh,
        out_type=jax.ShapeDtypeStruct((npad, c), jnp.float32),
        scratch_types=[
            pltpu.VMEM((2, ch), jnp.int32),
            pltpu.VMEM((2, ch, c), jnp.float32),
            pltpu.SemaphoreType.DMA,
            pltpu.SemaphoreType.DMA,
            pltpu.SemaphoreType.DMA,
        ],
    )
    def k(x_hbm, dest_hbm, out_hbm, idx_v, rows_v, in_sem0, in_sem1,
          out_sem):
        wid = lax.axis_index("s") * _SC_NC + lax.axis_index("c")
        base = wid * per_w
        in_sems = (in_sem0, in_sem1)

        def start_in(j):
            o = base + j * ch
            s = j % 2
            i1 = pltpu.make_async_copy(
                dest_hbm.at[pl.ds(o, ch)], idx_v.at[s], in_sems[s])
            i1.start()
            i2 = pltpu.make_async_copy(
                x_hbm.at[pl.ds(o, ch)], rows_v.at[s], in_sems[s])
            i2.start()
            return i1, i2

        pend_in = start_in(0)
        pend_out = None
        for j in range(n_ch):
            s = j % 2
            if pend_out is not None:
                pend_out.wait()
            nxt = start_in(j + 1) if j + 1 < n_ch else None
            pend_in[0].wait()
            pend_in[1].wait()
            cp = pltpu.make_async_copy(
                rows_v.at[s], out_hbm.at[idx_v.at[s]], out_sem)
            cp.start()
            pend_out = cp
            pend_in = nxt
        pend_out.wait()

    return k(xf, dest)


def _sc_gather(ys, dest, tok):
    _, c = ys.shape
    per_w = tok // _SC_NW
    ch = 64
    n_ch = per_w // ch
    mesh = plsc.VectorSubcoreMesh(core_axis_name="c", subcore_axis_name="s")

    @functools.partial(
        pl.kernel, mesh=mesh,
        out_type=jax.ShapeDtypeStruct((tok, c), jnp.float32),
        scratch_types=[
            pltpu.VMEM((2, ch), jnp.int32),
            pltpu.VMEM((2, ch, c), jnp.float32),
            pltpu.SemaphoreType.DMA,
            pltpu.SemaphoreType.DMA,
            pltpu.SemaphoreType.DMA,
            pltpu.SemaphoreType.DMA,
            pltpu.SemaphoreType.DMA,
        ],
    )
    def k(y_hbm, dest_hbm, out_hbm, idx_v, rows_v, idx_sem0, idx_sem1,
          g_sem, o_sem0, o_sem1):
        wid = lax.axis_index("s") * _SC_NC + lax.axis_index("c")
        base = wid * per_w
        idx_sems = (idx_sem0, idx_sem1)
        o_sems = (o_sem0, o_sem1)

        def start_idx(j):
            o = base + j * ch
            cp = pltpu.make_async_copy(
                dest_hbm.at[pl.ds(o, ch)], idx_v.at[j % 2], idx_sems[j % 2])
            cp.start()
            return cp

        def start_store(j):
            o = base + j * ch
            cp = pltpu.make_async_copy(
                rows_v.at[j % 2], out_hbm.at[pl.ds(o, ch)], o_sems[j % 2])
            cp.start()
            return cp

        pend_idx = start_idx(0)
        pend_gather = None
        pend_store = [None, None]
        for j in range(n_ch):
            s = j % 2
            if pend_gather is not None:
                pend_gather.wait()
                pend_store[1 - s] = start_store(j - 1)
            nxt_idx = start_idx(j + 1) if j + 1 < n_ch else None
            pend_idx.wait()
            if pend_store[s] is not None:
                pend_store[s].wait()
            g = pltpu.make_async_copy(
                y_hbm.at[idx_v.at[s]], rows_v.at[s], g_sem)
            g.start()
            pend_gather = g
            pend_idx = nxt_idx
        pend_gather.wait()
        last = n_ch - 1
        st = start_store(last)
        if pend_store[1 - (last % 2)] is not None:
            pend_store[1 - (last % 2)].wait()
        st.wait()

    return k(ys, dest)


def kernel(x, w_router, b_router, w_c_fc, b_c_fc, w_gate, b_gate, w_c_proj,
           b_c_proj):
    b, t, c = x.shape
    e, _, h = w_c_fc.shape
    tok = b * t
    n_rblk = tok // BLK_T
    nblk = tok // BM + e
    npad = nblk * BM
    xf = x.reshape(tok, c)

    tril = jnp.tril(jnp.ones((BLK_T, BLK_T), jnp.float32), -1)

    dest3, be = pl.pallas_call(
        _route_body,
        grid=(2, n_rblk),
        in_specs=[
            pl.BlockSpec((BLK_T, c), lambda p, i: (i * (1 - p), 0)),
            pl.BlockSpec((c, e), lambda p, i: (0, 0)),
            pl.BlockSpec((e,), lambda p, i: (0,)),
            pl.BlockSpec((BLK_T, BLK_T), lambda p, i: (0, 0)),
        ],
        out_specs=[
            pl.BlockSpec((1, 1, BLK_T), lambda p, i: (i, 0, 0)),
            pl.BlockSpec((1, 128), lambda p, i: (0, 0)),
        ],
        out_shape=[
            jax.ShapeDtypeStruct((n_rblk, 1, BLK_T), jnp.int32),
            jax.ShapeDtypeStruct((1, 128), jnp.int32),
        ],
        scratch_shapes=[
            pltpu.VMEM((n_rblk, BLK_T, e), jnp.float32),
            pltpu.VMEM((1, e), jnp.float32),
            pltpu.VMEM((1, e), jnp.float32),
        ],
    )(xf, w_router, b_router, tril)

    dest = dest3.reshape(tok)
    be_list = be.reshape(128)[:nblk]

    xs = _sc_scatter(xf, dest, npad)

    wix = lambda i, be: (jnp.minimum(be[i], e - 1), 0, 0)
    ys = pl.pallas_call(
        functools.partial(_gmm_body, e),
        grid_spec=pltpu.PrefetchScalarGridSpec(
            num_scalar_prefetch=1,
            grid=(nblk,),
            in_specs=[
                pl.BlockSpec((BM, c), lambda i, be: (i, 0)),
                pl.BlockSpec((1, c, h), wix),
                pl.BlockSpec((1, 1, h), wix),
                pl.BlockSpec((1, c, h), wix),
                pl.BlockSpec((1, 1, h), wix),
                pl.BlockSpec((1, h, c), wix),
                pl.BlockSpec((1, 1, c), wix),
            ],
            out_specs=pl.BlockSpec((BM, c), lambda i, be: (i, 0)),
        ),
        out_shape=jax.ShapeDtypeStruct((npad, c), jnp.float32),
    )(be_list, xs, w_c_fc, b_c_fc, w_gate, b_gate, w_c_proj, b_c_proj)

    yf = _sc_gather(ys, dest, tok)
    return yf.reshape(b, t, c)


# speedup vs baseline: 1.0876x; 1.0876x over previous
import functools

import jax
import jax.numpy as jnp
from jax import lax
from jax.experimental import pallas as pl
from jax.experimental.pallas import tpu as pltpu
from jax.experimental.pallas import tpu_sc as plsc

BLK_T = 512
BM = 512

_SC_NC = 2
_SC_NS = 16
_SC_NW = _SC_NC * _SC_NS


def _route_body(x_ref, wr_ref, br_ref, tril_ref, dest_ref, be_ref,
                oh_s, counts_s, running_s):
    p = pl.program_id(0)
    i = pl.program_id(1)

    @pl.when(p == 0)
    def _router():
        @pl.when(i == 0)
        def _():
            counts_s[...] = jnp.zeros_like(counts_s)
            running_s[...] = jnp.zeros_like(running_s)

        xb = x_ref[...]
        logits = jnp.dot(xb, wr_ref[...], preferred_element_type=jnp.float32)
        logits = logits + br_ref[...][None, :]
        n, e = logits.shape
        maxv = jnp.max(logits, axis=1, keepdims=True)
        ohm = (logits == maxv).astype(jnp.float32)
        tri8 = (lax.broadcasted_iota(jnp.int32, (e, e), 0)
                < lax.broadcasted_iota(jnp.int32, (e, e), 1)
                ).astype(jnp.float32)
        prior = jnp.dot(ohm, tri8, preferred_element_type=jnp.float32)
        oh = ohm * (prior == 0.0).astype(jnp.float32)
        oh_s[i] = oh
        counts_s[...] += jnp.sum(oh, axis=0, keepdims=True)

    @pl.when(p == 1)
    def _plan():
        counts = counts_s[...]
        e = counts.shape[1]
        pc = jnp.ceil(counts / BM) * BM
        tri = (lax.broadcasted_iota(jnp.int32, (e, e), 0)
               < lax.broadcasted_iota(jnp.int32, (e, e), 1)
               ).astype(jnp.float32)
        offs = jnp.dot(pc, tri, preferred_element_type=jnp.float32)
        ends = offs + pc

        oh = oh_s[i]
        earlier = jnp.dot(tril_ref[...], oh, preferred_element_type=jnp.float32)
        base = offs + running_s[...]
        destf = jnp.sum((base + earlier) * oh, axis=1)
        dest_ref[0, 0, :] = destf.astype(jnp.int32)
        running_s[...] += jnp.sum(oh, axis=0, keepdims=True)

        @pl.when(i == 0)
        def _():
            jb = lax.broadcasted_iota(jnp.int32, (1, 128), 1)
            jbf = jb.astype(jnp.float32) * BM
            lane8 = lax.broadcasted_iota(jnp.int32, (1, e), 1)
            acc = jnp.zeros((1, 128), jnp.float32)
            for ei in range(e):
                end_e = jnp.sum(jnp.where(lane8 == ei, ends, 0.0))
                acc += (jbf >= end_e).astype(jnp.float32)
            be_ref[...] = acc.astype(jnp.int32)


def _gmm_body(n_experts, be_ref, xs_ref, wfc_ref, bfc_ref, wg_ref, bg_ref,
              wp_ref, bp_ref, out_ref):
    i = pl.program_id(0)

    @pl.when(be_ref[i] < n_experts)
    def _():
        xb = xs_ref[...]
        h = jnp.dot(xb, wfc_ref[0], preferred_element_type=jnp.float32)
        h = h + bfc_ref[0]
        g = jnp.dot(xb, wg_ref[0], preferred_element_type=jnp.float32)
        g = g + bg_ref[0]
        g = g * (1.0 / (1.0 + jnp.exp(-g)))
        o = jnp.dot(h * g, wp_ref[0], preferred_element_type=jnp.float32)
        out_ref[...] = o + bp_ref[0]


def _sc_scatter(xf, dest, npad):
    tok, c = xf.shape
    per_w = tok // _SC_NW
    ch = 64
    n_ch = per_w // ch
    mesh = plsc.VectorSubcoreMesh(core_axis_name="c", subcore_axis_name="s")

    @functools.partial(
        pl.kernel, mesh=mesh,
        out_type=jax.ShapeDtypeStruct((npad, c), jnp.float32),
        scratch_types=[
            pltpu.VMEM((2, ch), jnp.int32),
            pltpu.VMEM((2, ch, c), jnp.float32),
            pltpu.SemaphoreType.DMA,
            pltpu.SemaphoreType.DMA,
            pltpu.SemaphoreType.DMA,
        ],
    )
    def k(x_hbm, dest_hbm, out_hbm, idx_v, rows_v, in_sem0, in_sem1,
          out_sem):
        wid = lax.axis_index("s") * _SC_NC + lax.axis_index("c")
        base = wid * per_w
        in_sems = (in_sem0, in_sem1)

        def start_in(j):
            o = base + j * ch
            s = j % 2
            i1 = pltpu.make_async_copy(
                dest_hbm.at[pl.ds(o, ch)], idx_v.at[s], in_sems[s])
            i1.start()
            i2 = pltpu.make_async_copy(
                x_hbm.at[pl.ds(o, ch)], rows_v.at[s], in_sems[s])
            i2.start()
            return i1, i2

        pend_in = start_in(0)
        pend_out = None
        for j in range(n_ch):
            s = j % 2
            if pend_out is not None:
                pend_out.wait()
            nxt = start_in(j + 1) if j + 1 < n_ch else None
            pend_in[0].wait()
            pend_in[1].wait()
            cp = pltpu.make_async_copy(
                rows_v.at[s], out_hbm.at[idx_v.at[s]], out_sem)
            cp.start()
            pend_out = cp
            pend_in = nxt
        pend_out.wait()

    return k(xf, dest)


def _sc_gather(ys, dest, tok):
    _, c = ys.shape
    per_w = tok // _SC_NW
    ch = 64
    n_ch = per_w // ch
    mesh = plsc.VectorSubcoreMesh(core_axis_name="c", subcore_axis_name="s")

    @functools.partial(
        pl.kernel, mesh=mesh,
        out_type=jax.ShapeDtypeStruct((tok, c), jnp.float32),
        scratch_types=[
            pltpu.VMEM((2, ch), jnp.int32),
            pltpu.VMEM((2, ch, c), jnp.float32),
            pltpu.SemaphoreType.DMA,
            pltpu.SemaphoreType.DMA,
            pltpu.SemaphoreType.DMA,
            pltpu.SemaphoreType.DMA,
            pltpu.SemaphoreType.DMA,
        ],
    )
    def k(y_hbm, dest_hbm, out_hbm, idx_v, rows_v, idx_sem0, idx_sem1,
          g_sem, o_sem0, o_sem1):
        wid = lax.axis_index("s") * _SC_NC + lax.axis_index("c")
        base = wid * per_w
        idx_sems = (idx_sem0, idx_sem1)
        o_sems = (o_sem0, o_sem1)

        def start_idx(j):
            o = base + j * ch
            cp = pltpu.make_async_copy(
                dest_hbm.at[pl.ds(o, ch)], idx_v.at[j % 2], idx_sems[j % 2])
            cp.start()
            return cp

        def start_store(j):
            o = base + j * ch
            cp = pltpu.make_async_copy(
                rows_v.at[j % 2], out_hbm.at[pl.ds(o, ch)], o_sems[j % 2])
            cp.start()
            return cp

        pend_idx = start_idx(0)
        pend_gather = None
        pend_store = [None, None]
        for j in range(n_ch):
            s = j % 2
            if pend_gather is not None:
                pend_gather.wait()
                pend_store[1 - s] = start_store(j - 1)
            nxt_idx = start_idx(j + 1) if j + 1 < n_ch else None
            pend_idx.wait()
            if pend_store[s] is not None:
                pend_store[s].wait()
            g = pltpu.make_async_copy(
                y_hbm.at[idx_v.at[s]], rows_v.at[s], g_sem)
            g.start()
            pend_gather = g
            pend_idx = nxt_idx
        pend_gather.wait()
        last = n_ch - 1
        st = start_store(last)
        if pend_store[1 - (last % 2)] is not None:
            pend_store[1 - (last % 2)].wait()
        st.wait()

    return k(ys, dest)


def kernel(x, w_router, b_router, w_c_fc, b_c_fc, w_gate, b_gate, w_c_proj,
           b_c_proj):
    b, t, c = x.shape
    e, _, h = w_c_fc.shape
    tok = b * t
    n_rblk = tok // BLK_T
    nblk = tok // BM + e
    npad = nblk * BM
    xf = x.reshape(tok, c)

    tril = jnp.tril(jnp.ones((BLK_T, BLK_T), jnp.float32), -1)

    dest3, be = pl.pallas_call(
        _route_body,
        grid=(2, n_rblk),
        in_specs=[
            pl.BlockSpec((BLK_T, c), lambda p, i: (i * (1 - p), 0)),
            pl.BlockSpec((c, e), lambda p, i: (0, 0)),
            pl.BlockSpec((e,), lambda p, i: (0,)),
            pl.BlockSpec((BLK_T, BLK_T), lambda p, i: (0, 0)),
        ],
        out_specs=[
            pl.BlockSpec((1, 1, BLK_T), lambda p, i: (i, 0, 0)),
            pl.BlockSpec((1, 128), lambda p, i: (0, 0)),
        ],
        out_shape=[
            jax.ShapeDtypeStruct((n_rblk, 1, BLK_T), jnp.int32),
            jax.ShapeDtypeStruct((1, 128), jnp.int32),
        ],
        scratch_shapes=[
            pltpu.VMEM((n_rblk, BLK_T, e), jnp.float32),
            pltpu.VMEM((1, e), jnp.float32),
            pltpu.VMEM((1, e), jnp.float32),
        ],
    )(xf, w_router, b_router, tril)

    dest = dest3.reshape(tok)
    be_list = be.reshape(128)[:nblk]

    xs = _sc_scatter(xf, dest, npad)

    wix = lambda i, be: (jnp.minimum(be[i], e - 1), 0, 0)
    ys = pl.pallas_call(
        functools.partial(_gmm_body, e),
        grid_spec=pltpu.PrefetchScalarGridSpec(
            num_scalar_prefetch=1,
            grid=(nblk,),
            in_specs=[
                pl.BlockSpec((BM, c), lambda i, be: (i, 0)),
                pl.BlockSpec((1, c, h), wix),
                pl.BlockSpec((1, 1, h), wix),
                pl.BlockSpec((1, c, h), wix),
                pl.BlockSpec((1, 1, h), wix),
                pl.BlockSpec((1, h, c), wix),
                pl.BlockSpec((1, 1, c), wix),
            ],
            out_specs=pl.BlockSpec((BM, c), lambda i, be: (i, 0)),
        ),
        out_shape=jax.ShapeDtypeStruct((npad, c), jnp.float32),
    )(be_list, xs, w_c_fc, b_c_fc, w_gate, b_gate, w_c_proj, b_c_proj)

    yf = _sc_gather(ys, dest, tok)
    return yf.reshape(b, t, c)


# speedup vs baseline: 1.0943x; 1.0061x over previous
import functools

import jax
import jax.numpy as jnp
from jax import lax
from jax.experimental import pallas as pl
from jax.experimental.pallas import tpu as pltpu
from jax.experimental.pallas import tpu_sc as plsc

BLK_T = 1024
BM = 512

_SC_NC = 2
_SC_NS = 16
_SC_NW = _SC_NC * _SC_NS


def _route_body(x_ref, wr_ref, br_ref, tril_ref, dest_ref, be_ref,
                oh_s, counts_s, running_s):
    p = pl.program_id(0)
    i = pl.program_id(1)

    @pl.when(p == 0)
    def _router():
        @pl.when(i == 0)
        def _():
            counts_s[...] = jnp.zeros_like(counts_s)
            running_s[...] = jnp.zeros_like(running_s)

        xb = x_ref[...]
        logits = jnp.dot(xb, wr_ref[...], preferred_element_type=jnp.float32)
        logits = logits + br_ref[...][None, :]
        n, e = logits.shape
        maxv = jnp.max(logits, axis=1, keepdims=True)
        ohm = (logits == maxv).astype(jnp.float32)
        tri8 = (lax.broadcasted_iota(jnp.int32, (e, e), 0)
                < lax.broadcasted_iota(jnp.int32, (e, e), 1)
                ).astype(jnp.float32)
        prior = jnp.dot(ohm, tri8, preferred_element_type=jnp.float32)
        oh = ohm * (prior == 0.0).astype(jnp.float32)
        oh_s[i] = oh
        counts_s[...] += jnp.sum(oh, axis=0, keepdims=True)

    @pl.when(p == 1)
    def _plan():
        counts = counts_s[...]
        e = counts.shape[1]
        pc = jnp.ceil(counts / BM) * BM
        tri = (lax.broadcasted_iota(jnp.int32, (e, e), 0)
               < lax.broadcasted_iota(jnp.int32, (e, e), 1)
               ).astype(jnp.float32)
        offs = jnp.dot(pc, tri, preferred_element_type=jnp.float32)
        ends = offs + pc

        oh = oh_s[i]
        earlier = jnp.dot(tril_ref[...], oh, preferred_element_type=jnp.float32)
        base = offs + running_s[...]
        destf = jnp.sum((base + earlier) * oh, axis=1)
        dest_ref[0, 0, :] = destf.astype(jnp.int32)
        running_s[...] += jnp.sum(oh, axis=0, keepdims=True)

        @pl.when(i == 0)
        def _():
            jb = lax.broadcasted_iota(jnp.int32, (1, 128), 1)
            jbf = jb.astype(jnp.float32) * BM
            lane8 = lax.broadcasted_iota(jnp.int32, (1, e), 1)
            acc = jnp.zeros((1, 128), jnp.float32)
            for ei in range(e):
                end_e = jnp.sum(jnp.where(lane8 == ei, ends, 0.0))
                acc += (jbf >= end_e).astype(jnp.float32)
            be_ref[...] = acc.astype(jnp.int32)


def _gmm_body(n_experts, be_ref, xs_ref, wfc_ref, bfc_ref, wg_ref, bg_ref,
              wp_ref, bp_ref, out_ref):
    i = pl.program_id(0)

    @pl.when(be_ref[i] < n_experts)
    def _():
        xb = xs_ref[...]
        h = jnp.dot(xb, wfc_ref[0], preferred_element_type=jnp.float32)
        h = h + bfc_ref[0]
        g = jnp.dot(xb, wg_ref[0], preferred_element_type=jnp.float32)
        g = g + bg_ref[0]
        g = g * (1.0 / (1.0 + jnp.exp(-g)))
        o = jnp.dot(h * g, wp_ref[0], preferred_element_type=jnp.float32)
        out_ref[...] = o + bp_ref[0]


def _sc_scatter(xf, dest, npad):
    tok, c = xf.shape
    per_w = tok // _SC_NW
    ch = 64
    n_ch = per_w // ch
    mesh = plsc.VectorSubcoreMesh(core_axis_name="c", subcore_axis_name="s")

    @functools.partial(
        pl.kernel, mesh=mesh,
        out_type=jax.ShapeDtypeStruct((npad, c), jnp.float32),
        scratch_types=[
            pltpu.VMEM((2, ch), jnp.int32),
            pltpu.VMEM((2, ch, c), jnp.float32),
            pltpu.SemaphoreType.DMA,
            pltpu.SemaphoreType.DMA,
            pltpu.SemaphoreType.DMA,
        ],
    )
    def k(x_hbm, dest_hbm, out_hbm, idx_v, rows_v, in_sem0, in_sem1,
          out_sem):
        wid = lax.axis_index("s") * _SC_NC + lax.axis_index("c")
        base = wid * per_w
        in_sems = (in_sem0, in_sem1)

        def start_in(j):
            o = base + j * ch
            s = j % 2
            i1 = pltpu.make_async_copy(
                dest_hbm.at[pl.ds(o, ch)], idx_v.at[s], in_sems[s])
            i1.start()
            i2 = pltpu.make_async_copy(
                x_hbm.at[pl.ds(o, ch)], rows_v.at[s], in_sems[s])
            i2.start()
            return i1, i2

        pend_in = start_in(0)
        pend_out = None
        for j in range(n_ch):
            s = j % 2
            if pend_out is not None:
                pend_out.wait()
            nxt = start_in(j + 1) if j + 1 < n_ch else None
            pend_in[0].wait()
            pend_in[1].wait()
            cp = pltpu.make_async_copy(
                rows_v.at[s], out_hbm.at[idx_v.at[s]], out_sem)
            cp.start()
            pend_out = cp
            pend_in = nxt
        pend_out.wait()

    return k(xf, dest)


def _sc_gather(ys, dest, tok):
    _, c = ys.shape
    per_w = tok // _SC_NW
    ch = 64
    n_ch = per_w // ch
    mesh = plsc.VectorSubcoreMesh(core_axis_name="c", subcore_axis_name="s")

    @functools.partial(
        pl.kernel, mesh=mesh,
        out_type=jax.ShapeDtypeStruct((tok, c), jnp.float32),
        scratch_types=[
            pltpu.VMEM((2, ch), jnp.int32),
            pltpu.VMEM((2, ch, c), jnp.float32),
            pltpu.SemaphoreType.DMA,
            pltpu.SemaphoreType.DMA,
            pltpu.SemaphoreType.DMA,
            pltpu.SemaphoreType.DMA,
            pltpu.SemaphoreType.DMA,
        ],
    )
    def k(y_hbm, dest_hbm, out_hbm, idx_v, rows_v, idx_sem0, idx_sem1,
          g_sem, o_sem0, o_sem1):
        wid = lax.axis_index("s") * _SC_NC + lax.axis_index("c")
        base = wid * per_w
        idx_sems = (idx_sem0, idx_sem1)
        o_sems = (o_sem0, o_sem1)

        def start_idx(j):
            o = base + j * ch
            cp = pltpu.make_async_copy(
                dest_hbm.at[pl.ds(o, ch)], idx_v.at[j % 2], idx_sems[j % 2])
            cp.start()
            return cp

        def start_store(j):
            o = base + j * ch
            cp = pltpu.make_async_copy(
                rows_v.at[j % 2], out_hbm.at[pl.ds(o, ch)], o_sems[j % 2])
            cp.start()
            return cp

        pend_idx = start_idx(0)
        pend_gather = None
        pend_store = [None, None]
        for j in range(n_ch):
            s = j % 2
            if pend_gather is not None:
                pend_gather.wait()
                pend_store[1 - s] = start_store(j - 1)
            nxt_idx = start_idx(j + 1) if j + 1 < n_ch else None
            pend_idx.wait()
            if pend_store[s] is not None:
                pend_store[s].wait()
            g = pltpu.make_async_copy(
                y_hbm.at[idx_v.at[s]], rows_v.at[s], g_sem)
            g.start()
            pend_gather = g
            pend_idx = nxt_idx
        pend_gather.wait()
        last = n_ch - 1
        st = start_store(last)
        if pend_store[1 - (last % 2)] is not None:
            pend_store[1 - (last % 2)].wait()
        st.wait()

    return k(ys, dest)


def kernel(x, w_router, b_router, w_c_fc, b_c_fc, w_gate, b_gate, w_c_proj,
           b_c_proj):
    b, t, c = x.shape
    e, _, h = w_c_fc.shape
    tok = b * t
    n_rblk = tok // BLK_T
    nblk = tok // BM + e
    npad = nblk * BM
    xf = x.reshape(tok, c)

    tril = jnp.tril(jnp.ones((BLK_T, BLK_T), jnp.float32), -1)

    dest3, be = pl.pallas_call(
        _route_body,
        grid=(2, n_rblk),
        in_specs=[
            pl.BlockSpec((BLK_T, c), lambda p, i: (i * (1 - p), 0)),
            pl.BlockSpec((c, e), lambda p, i: (0, 0)),
            pl.BlockSpec((e,), lambda p, i: (0,)),
            pl.BlockSpec((BLK_T, BLK_T), lambda p, i: (0, 0)),
        ],
        out_specs=[
            pl.BlockSpec((1, 1, BLK_T), lambda p, i: (i, 0, 0)),
            pl.BlockSpec((1, 128), lambda p, i: (0, 0)),
        ],
        out_shape=[
            jax.ShapeDtypeStruct((n_rblk, 1, BLK_T), jnp.int32),
            jax.ShapeDtypeStruct((1, 128), jnp.int32),
        ],
        scratch_shapes=[
            pltpu.VMEM((n_rblk, BLK_T, e), jnp.float32),
            pltpu.VMEM((1, e), jnp.float32),
            pltpu.VMEM((1, e), jnp.float32),
        ],
    )(xf, w_router, b_router, tril)

    dest = dest3.reshape(tok)
    be_list = be.reshape(128)[:nblk]

    xs = _sc_scatter(xf, dest, npad)

    wix = lambda i, be: (jnp.minimum(be[i], e - 1), 0, 0)
    ys = pl.pallas_call(
        functools.partial(_gmm_body, e),
        grid_spec=pltpu.PrefetchScalarGridSpec(
            num_scalar_prefetch=1,
            grid=(nblk,),
            in_specs=[
                pl.BlockSpec((BM, c), lambda i, be: (i, 0)),
                pl.BlockSpec((1, c, h), wix),
                pl.BlockSpec((1, 1, h), wix),
                pl.BlockSpec((1, c, h), wix),
                pl.BlockSpec((1, 1, h), wix),
                pl.BlockSpec((1, h, c), wix),
                pl.BlockSpec((1, 1, c), wix),
            ],
            out_specs=pl.BlockSpec((BM, c), lambda i, be: (i, 0)),
        ),
        out_shape=jax.ShapeDtypeStruct((npad, c), jnp.float32),
    )(be_list, xs, w_c_fc, b_c_fc, w_gate, b_gate, w_c_proj, b_c_proj)

    yf = _sc_gather(ys, dest, tok)
    return yf.reshape(b, t, c)


# speedup vs baseline: 1.1434x; 1.0449x over previous
import functools

import jax
import jax.numpy as jnp
from jax import lax
from jax.experimental import pallas as pl
from jax.experimental.pallas import tpu as pltpu
from jax.experimental.pallas import tpu_sc as plsc

BLK_T = 1024
BM = 512

_SC_NC = 2
_SC_NS = 16
_SC_NW = _SC_NC * _SC_NS


def _route_body(x_ref, wr_ref, br_ref, tril_ref, dest_ref, be_ref,
                oh_s, counts_s, running_s):
    p = pl.program_id(0)
    i = pl.program_id(1)

    @pl.when(p == 0)
    def _router():
        @pl.when(i == 0)
        def _():
            counts_s[...] = jnp.zeros_like(counts_s)
            running_s[...] = jnp.zeros_like(running_s)

        xb = x_ref[...]
        logits = jnp.dot(xb, wr_ref[...], preferred_element_type=jnp.float32)
        logits = logits + br_ref[...][None, :]
        n, e = logits.shape
        maxv = jnp.max(logits, axis=1, keepdims=True)
        ohm = (logits == maxv).astype(jnp.float32)
        tri8 = (lax.broadcasted_iota(jnp.int32, (e, e), 0)
                < lax.broadcasted_iota(jnp.int32, (e, e), 1)
                ).astype(jnp.float32)
        prior = jnp.dot(ohm, tri8, preferred_element_type=jnp.float32)
        oh = ohm * (prior == 0.0).astype(jnp.float32)
        oh_s[i] = oh
        counts_s[...] += jnp.sum(oh, axis=0, keepdims=True)

    @pl.when(p == 1)
    def _plan():
        counts = counts_s[...]
        e = counts.shape[1]
        pc = jnp.ceil(counts / BM) * BM
        tri = (lax.broadcasted_iota(jnp.int32, (e, e), 0)
               < lax.broadcasted_iota(jnp.int32, (e, e), 1)
               ).astype(jnp.float32)
        offs = jnp.dot(pc, tri, preferred_element_type=jnp.float32)
        ends = offs + pc

        oh = oh_s[i]
        earlier = jnp.dot(tril_ref[...], oh, preferred_element_type=jnp.float32)
        base = offs + running_s[...]
        destf = jnp.sum((base + earlier) * oh, axis=1, keepdims=True)
        dest_ref[0, :, :] = destf.astype(jnp.int32)
        running_s[...] += jnp.sum(oh, axis=0, keepdims=True)

        @pl.when(i == 0)
        def _():
            jb = lax.broadcasted_iota(jnp.int32, (1, 128), 1)
            jbf = jb.astype(jnp.float32) * BM
            lane8 = lax.broadcasted_iota(jnp.int32, (1, e), 1)
            acc = jnp.zeros((1, 128), jnp.float32)
            for ei in range(e):
                end_e = jnp.sum(jnp.where(lane8 == ei, ends, 0.0))
                acc += (jbf >= end_e).astype(jnp.float32)
            be_ref[...] = acc.astype(jnp.int32)


def _gmm_body(n_experts, be_ref, xs_ref, wfc_ref, bfc_ref, wg_ref, bg_ref,
              wp_ref, bp_ref, out_ref):
    i = pl.program_id(0)

    @pl.when(be_ref[i] < n_experts)
    def _():
        xb = xs_ref[...]
        h = jnp.dot(xb, wfc_ref[0], preferred_element_type=jnp.float32)
        h = h + bfc_ref[0]
        g = jnp.dot(xb, wg_ref[0], preferred_element_type=jnp.float32)
        g = g + bg_ref[0]
        g = g * (1.0 / (1.0 + jnp.exp(-g)))
        o = jnp.dot(h * g, wp_ref[0], preferred_element_type=jnp.float32)
        out_ref[...] = o + bp_ref[0]


def _sc_scatter(xf, dest, npad):
    tok, c = xf.shape
    per_w = tok // _SC_NW
    ch = 64
    n_ch = per_w // ch
    mesh = plsc.VectorSubcoreMesh(core_axis_name="c", subcore_axis_name="s")

    @functools.partial(
        pl.kernel, mesh=mesh,
        out_type=jax.ShapeDtypeStruct((npad, c), jnp.float32),
        scratch_types=[
            pltpu.VMEM((2, ch), jnp.int32),
            pltpu.VMEM((2, ch, c), jnp.float32),
            pltpu.SemaphoreType.DMA,
            pltpu.SemaphoreType.DMA,
            pltpu.SemaphoreType.DMA,
        ],
    )
    def k(x_hbm, dest_hbm, out_hbm, idx_v, rows_v, in_sem0, in_sem1,
          out_sem):
        wid = lax.axis_index("s") * _SC_NC + lax.axis_index("c")
        base = wid * per_w
        in_sems = (in_sem0, in_sem1)

        def start_in(j):
            o = base + j * ch
            s = j % 2
            i1 = pltpu.make_async_copy(
                dest_hbm.at[pl.ds(o, ch)], idx_v.at[s], in_sems[s])
            i1.start()
            i2 = pltpu.make_async_copy(
                x_hbm.at[pl.ds(o, ch)], rows_v.at[s], in_sems[s])
            i2.start()
            return i1, i2

        pend_in = start_in(0)
        pend_out = None
        for j in range(n_ch):
            s = j % 2
            if pend_out is not None:
                pend_out.wait()
            nxt = start_in(j + 1) if j + 1 < n_ch else None
            pend_in[0].wait()
            pend_in[1].wait()
            cp = pltpu.make_async_copy(
                rows_v.at[s], out_hbm.at[idx_v.at[s]], out_sem)
            cp.start()
            pend_out = cp
            pend_in = nxt
        pend_out.wait()

    return k(xf, dest)


def _sc_gather(ys, dest, tok):
    _, c = ys.shape
    per_w = tok // _SC_NW
    ch = 64
    n_ch = per_w // ch
    mesh = plsc.VectorSubcoreMesh(core_axis_name="c", subcore_axis_name="s")

    @functools.partial(
        pl.kernel, mesh=mesh,
        out_type=jax.ShapeDtypeStruct((tok, c), jnp.float32),
        scratch_types=[
            pltpu.VMEM((2, ch), jnp.int32),
            pltpu.VMEM((2, ch, c), jnp.float32),
            pltpu.SemaphoreType.DMA,
            pltpu.SemaphoreType.DMA,
            pltpu.SemaphoreType.DMA,
            pltpu.SemaphoreType.DMA,
            pltpu.SemaphoreType.DMA,
        ],
    )
    def k(y_hbm, dest_hbm, out_hbm, idx_v, rows_v, idx_sem0, idx_sem1,
          g_sem, o_sem0, o_sem1):
        wid = lax.axis_index("s") * _SC_NC + lax.axis_index("c")
        base = wid * per_w
        idx_sems = (idx_sem0, idx_sem1)
        o_sems = (o_sem0, o_sem1)

        def start_idx(j):
            o = base + j * ch
            cp = pltpu.make_async_copy(
                dest_hbm.at[pl.ds(o, ch)], idx_v.at[j % 2], idx_sems[j % 2])
            cp.start()
            return cp

        def start_store(j):
            o = base + j * ch
            cp = pltpu.make_async_copy(
                rows_v.at[j % 2], out_hbm.at[pl.ds(o, ch)], o_sems[j % 2])
            cp.start()
            return cp

        pend_idx = start_idx(0)
        pend_gather = None
        pend_store = [None, None]
        for j in range(n_ch):
            s = j % 2
            if pend_gather is not None:
                pend_gather.wait()
                pend_store[1 - s] = start_store(j - 1)
            nxt_idx = start_idx(j + 1) if j + 1 < n_ch else None
            pend_idx.wait()
            if pend_store[s] is not None:
                pend_store[s].wait()
            g = pltpu.make_async_copy(
                y_hbm.at[idx_v.at[s]], rows_v.at[s], g_sem)
            g.start()
            pend_gather = g
            pend_idx = nxt_idx
        pend_gather.wait()
        last = n_ch - 1
        st = start_store(last)
        if pend_store[1 - (last % 2)] is not None:
            pend_store[1 - (last % 2)].wait()
        st.wait()

    return k(ys, dest)


def kernel(x, w_router, b_router, w_c_fc, b_c_fc, w_gate, b_gate, w_c_proj,
           b_c_proj):
    b, t, c = x.shape
    e, _, h = w_c_fc.shape
    tok = b * t
    n_rblk = tok // BLK_T
    nblk = tok // BM + e
    npad = nblk * BM
    xf = x.reshape(tok, c)

    tril = jnp.tril(jnp.ones((BLK_T, BLK_T), jnp.float32), -1)

    dest3, be = pl.pallas_call(
        _route_body,
        grid=(2, n_rblk),
        in_specs=[
            pl.BlockSpec((BLK_T, c), lambda p, i: (i * (1 - p), 0)),
            pl.BlockSpec((c, e), lambda p, i: (0, 0)),
            pl.BlockSpec((e,), lambda p, i: (0,)),
            pl.BlockSpec((BLK_T, BLK_T), lambda p, i: (0, 0)),
        ],
        out_specs=[
            pl.BlockSpec((1, BLK_T, 1), lambda p, i: (i, 0, 0)),
            pl.BlockSpec((1, 128), lambda p, i: (0, 0)),
        ],
        out_shape=[
            jax.ShapeDtypeStruct((n_rblk, BLK_T, 1), jnp.int32),
            jax.ShapeDtypeStruct((1, 128), jnp.int32),
        ],
        scratch_shapes=[
            pltpu.VMEM((n_rblk, BLK_T, e), jnp.float32),
            pltpu.VMEM((1, e), jnp.float32),
            pltpu.VMEM((1, e), jnp.float32),
        ],
    )(xf, w_router, b_router, tril)

    dest = dest3.reshape(tok)
    be_list = be.reshape(128)[:nblk]

    xs = _sc_scatter(xf, dest, npad)

    wix = lambda i, be: (jnp.minimum(be[i], e - 1), 0, 0)
    ys = pl.pallas_call(
        functools.partial(_gmm_body, e),
        grid_spec=pltpu.PrefetchScalarGridSpec(
            num_scalar_prefetch=1,
            grid=(nblk,),
            in_specs=[
                pl.BlockSpec((BM, c), lambda i, be: (i, 0)),
                pl.BlockSpec((1, c, h), wix),
                pl.BlockSpec((1, 1, h), wix),
                pl.BlockSpec((1, c, h), wix),
                pl.BlockSpec((1, 1, h), wix),
                pl.BlockSpec((1, h, c), wix),
                pl.BlockSpec((1, 1, c), wix),
            ],
            out_specs=pl.BlockSpec((BM, c), lambda i, be: (i, 0)),
        ),
        out_shape=jax.ShapeDtypeStruct((npad, c), jnp.float32),
    )(be_list, xs, w_c_fc, b_c_fc, w_gate, b_gate, w_c_proj, b_c_proj)

    yf = _sc_gather(ys, dest, tok)
    return yf.reshape(b, t, c)
